# baseline (speedup 1.0000x reference)
import functools

import jax
import jax.numpy as jnp
from jax import lax
from jax.experimental import pallas as pl
from jax.experimental.pallas import tpu as pltpu
from jax.experimental.pallas import tpu_sc as plsc

B = 16384
D = 64
W = 128
NC, NS = 2, 16
NW = NC * NS
BPW = B // NW
BK = 1024


def _gather_body(user_idx, item_idx, pos_idx, neg_idx,
                 user_tab, item_tab, tag_tab,
                 u_out, i_out, p_out, n_out,
                 idx_v, rows_v, sem):
    wid = lax.axis_index("s") * NC + lax.axis_index("c")
    base = wid * BPW
    jobs = ((user_idx, user_tab, u_out),
            (item_idx, item_tab, i_out),
            (pos_idx, tag_tab, p_out),
            (neg_idx, tag_tab, n_out))
    for idx_hbm, tab, out in jobs:
        pltpu.sync_copy(idx_hbm.at[pl.ds(base, BPW)], idx_v)
        for k in range(BPW // 16):
            sl = pl.ds(k * 16, 16)
            idx_v[sl] = lax.shift_right_logical(idx_v[sl], 1)
        pltpu.async_copy(tab.at[idx_v], rows_v, sem).wait()
        pltpu.sync_copy(rows_v, out.at[pl.ds(base, BPW)])


@jax.jit
def _gather(user, item, pos_tag, neg_tag, user_tab2, item_tab2, tag_tab2):
    mesh = plsc.VectorSubcoreMesh(core_axis_name="c", subcore_axis_name="s",
                                  num_cores=NC, num_subcores=NS)
    emb = jax.ShapeDtypeStruct((B, W), jnp.float32)
    run = pl.kernel(
        _gather_body,
        out_type=(emb, emb, emb, emb),
        mesh=mesh,
        scratch_types=[
            pltpu.VMEM((BPW,), jnp.int32),
            pltpu.VMEM((BPW, W), jnp.float32),
            pltpu.SemaphoreType.DMA,
        ],
    )
    return run(user, item, pos_tag, neg_tag, user_tab2, item_tab2, tag_tab2)


def _sel(x2, par):
    return x2[:, :D] * (1.0 - par) + x2[:, D:] * par


def _score_body(u_ref, i_ref, p_ref, n_ref, pu_ref, pi_ref, pp_ref, pn_ref,
                e_ref, c_ref, out_ref):
    u_emb = _sel(u_ref[...], pu_ref[...])
    i_emb = _sel(i_ref[...], pi_ref[...])
    u_bf = u_emb.astype(jnp.bfloat16)
    i_bf = i_emb.astype(jnp.bfloat16)
    u_exp = jnp.dot(u_bf, e_ref[...],
                    preferred_element_type=jnp.float32).astype(jnp.bfloat16)
    i_tiled = pltpu.repeat(i_bf, D, axis=1)
    p_outer = u_exp * i_tiled
    w = jnp.dot(p_outer, c_ref[...], preferred_element_type=jnp.float32)
    d = _sel(p_ref[...], pp_ref[...]) - _sel(n_ref[...], pn_ref[...])
    s = jnp.sum(w * d, axis=1)
    ls = jnp.minimum(s, 0.0) - jnp.log(1.0 + jnp.exp(-jnp.abs(s)))
    part = jnp.sum(ls)

    @pl.when(pl.program_id(0) == 0)
    def _init():
        out_ref[0, 0] = 0.0

    out_ref[0, 0] += part

    @pl.when(pl.program_id(0) == pl.num_programs(0) - 1)
    def _fin():
        out_ref[0, 0] = out_ref[0, 0] * (-1.0 / B)


@jax.jit
def _score(u2, i2, p2, n2, pu, pi, pp, pn, core_tensor):
    e_mat = jnp.repeat(jnp.eye(D, dtype=jnp.bfloat16), D, axis=1)
    c_mat = core_tensor.reshape(D * D, D).astype(jnp.bfloat16)
    row = pl.BlockSpec((BK, W), lambda i: (i, 0))
    col = pl.BlockSpec((BK, 1), lambda i: (i, 0))
    loss = pl.pallas_call(
        _score_body,
        grid=(B // BK,),
        in_specs=[
            row, row, row, row,
            col, col, col, col,
            pl.BlockSpec((D, D * D), lambda i: (0, 0)),
            pl.BlockSpec((D * D, D), lambda i: (0, 0)),
        ],
        out_specs=pl.BlockSpec((1, 1), lambda i: (0, 0),
                               memory_space=pltpu.SMEM),
        out_shape=jax.ShapeDtypeStruct((1, 1), jnp.float32),
    )(u2, i2, p2, n2, pu, pi, pp, pn, e_mat, c_mat)
    return loss[0, 0]


def kernel(user, item, pos_tag, neg_tag, user_table, item_table,
           good_tag_table, core_tensor):
    ut2 = user_table.reshape(-1, W)
    it2 = item_table.reshape(-1, W)
    tt2 = good_tag_table.reshape(-1, W)
    u2, i2, p2, n2 = _gather(user, item, pos_tag, neg_tag, ut2, it2, tt2)
    par = lambda idx: (idx % 2).astype(jnp.float32).reshape(B, 1)
    return _score(u2, i2, p2, n2, par(user), par(item), par(pos_tag),
                  par(neg_tag), core_tensor)
